# baseline (speedup 1.0000x reference)
import jax
import jax.numpy as jnp
from jax.experimental import pallas as pl

_B = 256
_NPG = 30
_EPG = 96
_MAXE = 128
_NG = 2 * _B
_NODE_FEAT = 16
_STATE = 32
_MSG = 32
_PROP = 3
_TIME = 2
_SINK_ITERS = 10
_TEMP = 0.1

_NP = 32
_P = 8
_G = 2 * _P
_EB = _G * _EPG
_NB = _G * _NP
_F32 = jnp.float32


def _mm(a, b):
    return jax.lax.dot_general(a, b, (((1,), (0,)), ((), ())),
                               preferred_element_type=_F32)


def _bmm(a, b):
    return jax.lax.dot_general(a, b, (((2,), (1,)), ((0,), (0,))),
                               preferred_element_type=_F32)


def _body(nf_ref, ef_ref, lf_ref, lt_ref,
          enw, enb, eew, eeb,
          m1w, m1b, m2w, m2b,
          u1w, u1b, u2w, u2b,
          i1w, i1b, i2w, i2b,
          s1w, s1b, s2w, s2b,
          out_ref):
    relu = lambda x: jnp.maximum(x, 0.0)

    lf = lf_ref[...].reshape(_G, _EPG, 1)
    lt = lt_ref[...].reshape(_G, _EPG, 1)
    i2n = jax.lax.broadcasted_iota(jnp.int32, (_G, _EPG, 2 * _NP), 2)
    s_cat = jnp.logical_or(lf == i2n, lt + _NP == i2n).astype(_F32)
    i1n = jax.lax.broadcasted_iota(jnp.int32, (_G, _EPG, _NP), 2)
    s_to = (lt == i1n).astype(_F32)

    h0 = _mm(nf_ref[...], enw[...]) + enb[...]
    e0 = _mm(ef_ref[...], eew[...]) + eeb[...]

    m1w_v = m1w[...]
    w1h = jnp.concatenate([m1w_v[0:_STATE, :], m1w_v[_STATE:2 * _STATE, :]],
                          axis=1)
    wc = _mm(i2w[...], m1w_v[2 * _STATE:, :])
    bc = _mm(i2b[...], m1w_v[2 * _STATE:, :]) + m1b[...]
    i1w_v = i1w[...]
    m2w_v = m2w[...]
    m2b_v = m2b[...]

    def edge_pre(h, c_c):
        hab = _mm(h, w1h)
        h_a = hab[:, 0:2 * _STATE].reshape(_G, _NP, 2 * _STATE)
        h_b = hab[:, 2 * _STATE:].reshape(_G, _NP, 2 * _STATE)
        h_catted = jnp.concatenate([h_a, h_b], axis=1)
        gath = _bmm(s_cat, h_catted).reshape(_EB, 2 * _STATE)
        return relu(gath + c_c)

    transport_plan = None
    qs = cs = None
    for _t in range(_TIME):
        h = h0
        efe = e0
        inter = None
        for _s in range(_PROP):
            if inter is None:
                x1 = _mm(efe, i1w_v[0:_MSG, :])
            else:
                x1 = _mm(efe, i1w_v[0:_MSG, :]) + _mm(inter, i1w_v[_MSG:, :])
            c_c = _mm(relu(x1 + i1b[...]), wc) + bc

            r1 = jnp.concatenate(
                [edge_pre(h, c_c), jnp.ones((_EB, 1), _F32)], axis=1)
            aggc = jax.lax.dot_general(
                s_to, r1.reshape(_G, _EPG, 2 * _STATE + 1),
                (((1,), (1,)), ((0,), (0,))),
                preferred_element_type=_F32).reshape(_NB, 2 * _STATE + 1)
            agg = (_mm(aggc[:, 0:2 * _STATE], m2w_v)
                   + aggc[:, 2 * _STATE:] * m2b_v)
            hu = jnp.concatenate([h, agg], axis=1)
            h = _mm(relu(_mm(hu, u1w[...]) + u1b[...]), u2w[...]) + u2b[...]
            efe = _mm(edge_pre(h, c_c), m2w_v) + m2b_v

            if transport_plan is not None and _s + 1 < _PROP:
                efe4 = efe.reshape(_P, 2, _EPG, _MSG)
                zpad = jnp.zeros((_P, _MAXE - _EPG, _MSG), _F32)
                qp = jnp.concatenate([efe4[:, 0], zpad], axis=1)
                cp = jnp.concatenate([efe4[:, 1], zpad], axis=1)
                qi = _bmm(transport_plan, cp)
                ci = jax.lax.dot_general(
                    transport_plan, qp, (((1,), (1,)), ((0,), (0,))),
                    preferred_element_type=_F32)
                inter = jnp.concatenate(
                    [qi[:, 0:_EPG, :].reshape(_P, 1, _EPG, _MSG),
                     ci[:, 0:_EPG, :].reshape(_P, 1, _EPG, _MSG)],
                    axis=1).reshape(_EB, _MSG)

        efe4 = efe.reshape(_P, 2, _EPG, _MSG)
        zpad = jnp.zeros((_P, _MAXE - _EPG, _MSG), _F32)
        qs = jnp.concatenate([efe4[:, 0], zpad], axis=1)
        cs = jnp.concatenate([efe4[:, 1], zpad], axis=1)

        def sink_mlp(x):
            y = _mm(relu(_mm(x.reshape(_P * _MAXE, _MSG), s1w[...]) + s1b[...]),
                    s2w[...]) + s2b[...]
            return y.reshape(_P, _MAXE, _MSG)

        rowmask = (jax.lax.broadcasted_iota(jnp.int32, (_P, _MAXE, 1), 1)
                   < _EPG).astype(_F32)
        tq = sink_mlp(qs) * rowmask
        tc = sink_mlp(cs) * rowmask
        la0 = jax.lax.dot_general(
            tq, tc, (((2,), (2,)), ((0,), (0,))),
            preferred_element_type=_F32) * (1.0 / _TEMP)
        g = jnp.zeros((_P, 1, _MAXE), _F32)
        f = None
        for _i in range(_SINK_ITERS):
            x = la0 - g
            m = jnp.max(x, axis=2, keepdims=True)
            f = m + jnp.log(jnp.sum(jnp.exp(x - m), axis=2, keepdims=True))
            x = la0 - f
            m = jnp.max(x, axis=1, keepdims=True)
            g = m + jnp.log(jnp.sum(jnp.exp(x - m), axis=1, keepdims=True))
        transport_plan = jnp.exp(la0 - f - g)

    tpc = _bmm(transport_plan, cs)
    score = -jnp.sum(relu(qs - tpc), axis=(1, 2))
    out_ref[...] = score.reshape(_P, 1)


def kernel(node_features, edge_features, params, from_idx, to_idx):
    p = params
    n_feat = node_features.shape[1]

    nfp = jnp.pad(node_features.reshape(_NG, _NPG, n_feat),
                  ((0, 0), (0, _NP - _NPG), (0, 0))).reshape(_NG * _NP, n_feat)
    base = (jnp.arange(_NG, dtype=jnp.int32) * _NPG)[:, None]
    lf = from_idx.reshape(_NG, _EPG) - base
    lt = to_idx.reshape(_NG, _EPG) - base

    def b2(b):
        return b.reshape(1, -1)

    weights = [
        p['enc_node'][0], b2(p['enc_node'][1]),
        p['enc_edge'][0], b2(p['enc_edge'][1]),
        p['msg1'][0], b2(p['msg1'][1]),
        p['msg2'][0], b2(p['msg2'][1]),
        p['upd1'][0], b2(p['upd1'][1]),
        p['upd2'][0], b2(p['upd2'][1]),
        p['int1'][0], b2(p['int1'][1]),
        p['int2'][0], b2(p['int2'][1]),
        p['sink1'][0], b2(p['sink1'][1]),
        p['sink2'][0], b2(p['sink2'][1]),
    ]

    grid = _NG // _G
    w_specs = [pl.BlockSpec(w.shape, lambda i: (0, 0)) for w in weights]
    out = pl.pallas_call(
        _body,
        grid=(grid,),
        in_specs=[
            pl.BlockSpec((_NB, n_feat), lambda i: (i, 0)),
            pl.BlockSpec((_EB, edge_features.shape[1]), lambda i: (i, 0)),
            pl.BlockSpec((_G, _EPG), lambda i: (i, 0)),
            pl.BlockSpec((_G, _EPG), lambda i: (i, 0)),
        ] + w_specs,
        out_specs=pl.BlockSpec((_P, 1), lambda i: (i, 0)),
        out_shape=jax.ShapeDtypeStruct((_B, 1), _F32),
    )(nfp, edge_features, lf, lt, *weights)
    return out.reshape(_B)


# speedup vs baseline: 1.0353x; 1.0353x over previous
import jax
import jax.numpy as jnp
from jax.experimental import pallas as pl

_B = 256
_NPG = 30
_EPG = 96
_MAXE = 128
_NG = 2 * _B
_NODE_FEAT = 16
_STATE = 32
_MSG = 32
_PROP = 3
_TIME = 2
_SINK_ITERS = 10
_TEMP = 0.1

_NP = 32
_P = 8
_G = 2 * _P
_EB = _G * _EPG
_NB = _G * _NP
_F32 = jnp.float32
_LOG2E = 1.4426950408889634


def _mm(a, b):
    return jax.lax.dot_general(a, b, (((1,), (0,)), ((), ())),
                               preferred_element_type=_F32)


_mm32 = _mm


def _bmm(a, b):
    return jax.lax.dot_general(a, b, (((2,), (1,)), ((0,), (0,))),
                               preferred_element_type=_F32)


def _body(nf_ref, ef_ref, lf_ref, lt_ref,
          enw, enb, eew, eeb,
          m1w, m1b, m2w, m2b,
          u1w, u1b, u2w, u2b,
          i1w, i1b, i2w, i2b,
          s1w, s1b, s2w, s2b,
          out_ref):
    relu = lambda x: jnp.maximum(x, 0.0)

    lf = lf_ref[...].reshape(_G, _EPG, 1)
    lt = lt_ref[...].reshape(_G, _EPG, 1)
    i2n = jax.lax.broadcasted_iota(jnp.int32, (_G, _EPG, 2 * _NP), 2)
    s_cat = jnp.logical_or(lf == i2n, lt + _NP == i2n).astype(_F32)
    i1n = jax.lax.broadcasted_iota(jnp.int32, (_G, _EPG, _NP), 2)
    s_to = (lt == i1n).astype(_F32)

    h0 = _mm(nf_ref[...], enw[...]) + enb[...]
    e0 = _mm(ef_ref[...], eew[...]) + eeb[...]

    m1w_v = m1w[...]
    w1h = jnp.concatenate([m1w_v[0:_STATE, :], m1w_v[_STATE:2 * _STATE, :]],
                          axis=1)
    wc = _mm32(i2w[...], m1w_v[2 * _STATE:, :])
    bc = _mm32(i2b[...], m1w_v[2 * _STATE:, :]) + m1b[...]
    i1w_v = i1w[...]
    m2w_v = m2w[...]
    m2b_v = m2b[...]

    def edge_pre(h, c_c):
        hab = _mm(h, w1h)
        h_a = hab[:, 0:2 * _STATE].reshape(_G, _NP, 2 * _STATE)
        h_b = hab[:, 2 * _STATE:].reshape(_G, _NP, 2 * _STATE)
        h_catted = jnp.concatenate([h_a, h_b], axis=1)
        gath = _bmm(s_cat, h_catted).reshape(_EB, 2 * _STATE)
        return relu(gath + c_c)

    transport_plan = None
    qs = cs = None
    for _t in range(_TIME):
        h = h0
        efe = e0
        inter = None
        for _s in range(_PROP):
            if inter is None:
                x1 = _mm(efe, i1w_v[0:_MSG, :])
            else:
                x1 = _mm(efe, i1w_v[0:_MSG, :]) + _mm(inter, i1w_v[_MSG:, :])
            c_c = _mm(relu(x1 + i1b[...]), wc) + bc

            r1 = jnp.concatenate(
                [edge_pre(h, c_c), jnp.ones((_EB, 1), _F32)], axis=1)
            aggc = jax.lax.dot_general(
                s_to,
                r1.reshape(_G, _EPG, 2 * _STATE + 1),
                (((1,), (1,)), ((0,), (0,))),
                preferred_element_type=_F32).reshape(_NB, 2 * _STATE + 1)
            agg = (_mm(aggc[:, 0:2 * _STATE], m2w_v)
                   + aggc[:, 2 * _STATE:] * m2b_v)
            hu = jnp.concatenate([h, agg], axis=1)
            h = _mm(relu(_mm(hu, u1w[...]) + u1b[...]), u2w[...]) + u2b[...]
            efe = _mm(edge_pre(h, c_c), m2w_v) + m2b_v

            if transport_plan is not None and _s + 1 < _PROP:
                efe4 = efe.reshape(_P, 2, _EPG, _MSG)
                zpad = jnp.zeros((_P, _MAXE - _EPG, _MSG), _F32)
                qp = jnp.concatenate([efe4[:, 0], zpad], axis=1)
                cp = jnp.concatenate([efe4[:, 1], zpad], axis=1)
                qi = _bmm(transport_plan, cp)
                ci = jax.lax.dot_general(
                    transport_plan, qp,
                    (((1,), (1,)), ((0,), (0,))),
                    preferred_element_type=_F32)
                inter = jnp.concatenate(
                    [qi[:, 0:_EPG, :].reshape(_P, 1, _EPG, _MSG),
                     ci[:, 0:_EPG, :].reshape(_P, 1, _EPG, _MSG)],
                    axis=1).reshape(_EB, _MSG)

        efe4 = efe.reshape(_P, 2, _EPG, _MSG)
        zpad = jnp.zeros((_P, _MAXE - _EPG, _MSG), _F32)
        qs = jnp.concatenate([efe4[:, 0], zpad], axis=1)
        cs = jnp.concatenate([efe4[:, 1], zpad], axis=1)

        def sink_mlp(x):
            y = _mm32(relu(_mm32(x.reshape(_P * _MAXE, _MSG), s1w[...])
                           + s1b[...]), s2w[...]) + s2b[...]
            return y.reshape(_P, _MAXE, _MSG)

        rowmask = (jax.lax.broadcasted_iota(jnp.int32, (_P, _MAXE, 1), 1)
                   < _EPG).astype(_F32)
        tq = sink_mlp(qs) * rowmask
        tc = sink_mlp(cs) * rowmask
        la0 = jax.lax.dot_general(
            tq, tc, (((2,), (2,)), ((0,), (0,))),
            preferred_element_type=_F32) * (_LOG2E / _TEMP)
        ones_col = jnp.ones((_MAXE, 1), _F32)
        g = jnp.zeros((_P, 1, _MAXE), _F32)
        f = jnp.max(la0, axis=2, keepdims=True)
        for _i in range(_SINK_ITERS):
            e1 = jnp.exp2(la0 - g - f)
            s1 = jax.lax.dot_general(
                e1, ones_col, (((2,), (0,)), ((), ())),
                preferred_element_type=_F32)
            f = f + jnp.log2(s1)
            e2 = jnp.exp2(la0 - f - g)
            s2 = jnp.sum(e2, axis=1, keepdims=True)
            g = g + jnp.log2(s2)
        transport_plan = e2 * (1.0 / s2)

    tpc = jax.lax.dot_general(
        transport_plan, cs, (((2,), (1,)), ((0,), (0,))),
        preferred_element_type=_F32)
    score = -jnp.sum(relu(qs - tpc), axis=(1, 2))
    out_ref[...] = score.reshape(_P, 1)


def kernel(node_features, edge_features, params, from_idx, to_idx):
    p = params
    n_feat = node_features.shape[1]

    nfp = jnp.pad(node_features.reshape(_NG, _NPG, n_feat),
                  ((0, 0), (0, _NP - _NPG), (0, 0))).reshape(_NG * _NP, n_feat)
    base = (jnp.arange(_NG, dtype=jnp.int32) * _NPG)[:, None]
    lf = from_idx.reshape(_NG, _EPG) - base
    lt = to_idx.reshape(_NG, _EPG) - base

    def b2(b):
        return b.reshape(1, -1)

    weights = [
        p['enc_node'][0], b2(p['enc_node'][1]),
        p['enc_edge'][0], b2(p['enc_edge'][1]),
        p['msg1'][0], b2(p['msg1'][1]),
        p['msg2'][0], b2(p['msg2'][1]),
        p['upd1'][0], b2(p['upd1'][1]),
        p['upd2'][0], b2(p['upd2'][1]),
        p['int1'][0], b2(p['int1'][1]),
        p['int2'][0], b2(p['int2'][1]),
        p['sink1'][0], b2(p['sink1'][1]),
        p['sink2'][0], b2(p['sink2'][1]),
    ]

    grid = _NG // _G
    w_specs = [pl.BlockSpec(w.shape, lambda i: (0, 0)) for w in weights]
    out = pl.pallas_call(
        _body,
        grid=(grid,),
        in_specs=[
            pl.BlockSpec((_NB, n_feat), lambda i: (i, 0)),
            pl.BlockSpec((_EB, edge_features.shape[1]), lambda i: (i, 0)),
            pl.BlockSpec((_G, _EPG), lambda i: (i, 0)),
            pl.BlockSpec((_G, _EPG), lambda i: (i, 0)),
        ] + w_specs,
        out_specs=pl.BlockSpec((_P, 1), lambda i: (i, 0)),
        out_shape=jax.ShapeDtypeStruct((_B, 1), _F32),
    )(nfp, edge_features, lf, lt, *weights)
    return out.reshape(_B)


# speedup vs baseline: 1.0863x; 1.0494x over previous
import jax
import jax.numpy as jnp
from jax.experimental import pallas as pl

_B = 256
_NPG = 30
_EPG = 96
_MAXE = 128
_NG = 2 * _B
_NODE_FEAT = 16
_STATE = 32
_MSG = 32
_PROP = 3
_TIME = 2
_SINK_ITERS = 10
_TEMP = 0.1

_NP = 32
_P = 8
_G = 2 * _P
_EB = _G * _EPG
_NB = _G * _NP
_F32 = jnp.float32
_LOG2E = 1.4426950408889634


def _mm(a, b):
    return jax.lax.dot_general(a, b, (((1,), (0,)), ((), ())),
                               preferred_element_type=_F32)


_mm32 = _mm


def _bmm(a, b):
    return jax.lax.dot_general(a, b, (((2,), (1,)), ((0,), (0,))),
                               preferred_element_type=_F32)


def _body(nf_ref, ef_ref, lf_ref, lt_ref,
          enw, enb, eew, eeb,
          m1w, m1b, m2w, m2b,
          u1w, u1b, u2w, u2b,
          i1w, i1b, i2w, i2b,
          s1w, s1b, s2w, s2b,
          out_ref):
    relu = lambda x: jnp.maximum(x, 0.0)

    lf = lf_ref[...].reshape(_G, _EPG, 1)
    lt = lt_ref[...].reshape(_G, _EPG, 1)
    i2n = jax.lax.broadcasted_iota(jnp.int32, (_G, _EPG, 2 * _NP), 2)
    s_cat = jnp.logical_or(lf == i2n, lt + _NP == i2n).astype(_F32)
    i1n = jax.lax.broadcasted_iota(jnp.int32, (_G, _EPG, _NP), 2)
    s_to = (lt == i1n).astype(_F32)

    h0 = _mm(nf_ref[...], enw[...]) + enb[...]
    e0 = _mm(ef_ref[...], eew[...]) + eeb[...]

    m1w_v = m1w[...]
    w1h = jnp.concatenate([m1w_v[0:_STATE, :], m1w_v[_STATE:2 * _STATE, :]],
                          axis=1)
    wc = _mm32(i2w[...], m1w_v[2 * _STATE:, :])
    bc = _mm32(i2b[...], m1w_v[2 * _STATE:, :]) + m1b[...]
    i1w_v = i1w[...]
    m2w_v = m2w[...]
    m2b_v = m2b[...]

    def edge_pre(h, c_c):
        hab = _mm(h, w1h)
        h_a = hab[:, 0:2 * _STATE].reshape(_G, _NP, 2 * _STATE)
        h_b = hab[:, 2 * _STATE:].reshape(_G, _NP, 2 * _STATE)
        h_catted = jnp.concatenate([h_a, h_b], axis=1)
        gath = _bmm(s_cat, h_catted).reshape(_EB, 2 * _STATE)
        return relu(gath + c_c)

    transport_plan = None
    qs = cs = None
    for _t in range(_TIME):
        h = h0
        efe = e0
        inter = None
        for _s in range(_PROP):
            if inter is None:
                x1 = _mm(efe, i1w_v[0:_MSG, :])
            else:
                x1 = _mm(efe, i1w_v[0:_MSG, :]) + _mm(inter, i1w_v[_MSG:, :])
            c_c = _mm(relu(x1 + i1b[...]), wc) + bc

            r1 = jnp.concatenate(
                [edge_pre(h, c_c), jnp.ones((_EB, 1), _F32)], axis=1)
            aggc = jax.lax.dot_general(
                s_to,
                r1.reshape(_G, _EPG, 2 * _STATE + 1),
                (((1,), (1,)), ((0,), (0,))),
                preferred_element_type=_F32).reshape(_NB, 2 * _STATE + 1)
            agg = (_mm(aggc[:, 0:2 * _STATE], m2w_v)
                   + aggc[:, 2 * _STATE:] * m2b_v)
            hu = jnp.concatenate([h, agg], axis=1)
            h = _mm(relu(_mm(hu, u1w[...]) + u1b[...]), u2w[...]) + u2b[...]
            efe = _mm(edge_pre(h, c_c), m2w_v) + m2b_v

            if transport_plan is not None and _s + 1 < _PROP:
                efe4 = efe.reshape(_P, 2, _EPG, _MSG)
                zpad = jnp.zeros((_P, _MAXE - _EPG, _MSG), _F32)
                qp = jnp.concatenate([efe4[:, 0], zpad], axis=1)
                cp = jnp.concatenate([efe4[:, 1], zpad], axis=1)
                qi = _bmm(transport_plan, cp)
                ci = jax.lax.dot_general(
                    transport_plan, qp,
                    (((1,), (1,)), ((0,), (0,))),
                    preferred_element_type=_F32)
                inter = jnp.concatenate(
                    [qi[:, 0:_EPG, :].reshape(_P, 1, _EPG, _MSG),
                     ci[:, 0:_EPG, :].reshape(_P, 1, _EPG, _MSG)],
                    axis=1).reshape(_EB, _MSG)

        efe4 = efe.reshape(_P, 2, _EPG, _MSG)
        zpad = jnp.zeros((_P, _MAXE - _EPG, _MSG), _F32)
        qs = jnp.concatenate([efe4[:, 0], zpad], axis=1)
        cs = jnp.concatenate([efe4[:, 1], zpad], axis=1)

        def sink_mlp(x):
            y = _mm32(relu(_mm32(x.reshape(_P * _MAXE, _MSG), s1w[...])
                           + s1b[...]), s2w[...]) + s2b[...]
            return y.reshape(_P, _MAXE, _MSG)

        rowmask = (jax.lax.broadcasted_iota(jnp.int32, (_P, _MAXE, 1), 1)
                   < _EPG).astype(_F32)
        tq = sink_mlp(qs) * rowmask
        tc = sink_mlp(cs) * rowmask
        la0 = jax.lax.dot_general(
            tq, tc, (((2,), (2,)), ((0,), (0,))),
            preferred_element_type=_F32) * (_LOG2E / _TEMP)
        ones_col = jnp.ones((_MAXE, 1), _F32)
        f = jnp.max(la0, axis=2, keepdims=True)
        e1 = jnp.exp2(la0 - f)
        s1 = jax.lax.dot_general(
            e1, ones_col, (((2,), (0,)), ((), ())),
            preferred_element_type=_F32)
        f = f + jnp.log2(s1)
        x2 = la0 - f
        m2 = jnp.max(x2, axis=1, keepdims=True)
        e2 = jnp.exp2(x2 - m2)
        s2 = jnp.sum(e2, axis=1, keepdims=True)
        r2 = 1.0 / s2
        for _i in range(_SINK_ITERS - 1):
            e1 = e2 * r2
            s1 = jax.lax.dot_general(
                e1, ones_col, (((2,), (0,)), ((), ())),
                preferred_element_type=_F32)
            e2 = e1 * (1.0 / s1)
            s2 = jnp.sum(e2, axis=1, keepdims=True)
            r2 = 1.0 / s2
        transport_plan = e2 * r2

    tpc = jax.lax.dot_general(
        transport_plan, cs, (((2,), (1,)), ((0,), (0,))),
        preferred_element_type=_F32)
    score = -jnp.sum(relu(qs - tpc), axis=(1, 2))
    out_ref[...] = score.reshape(_P, 1)


def kernel(node_features, edge_features, params, from_idx, to_idx):
    p = params
    n_feat = node_features.shape[1]

    nfp = jnp.pad(node_features.reshape(_NG, _NPG, n_feat),
                  ((0, 0), (0, _NP - _NPG), (0, 0))).reshape(_NG * _NP, n_feat)
    base = (jnp.arange(_NG, dtype=jnp.int32) * _NPG)[:, None]
    lf = from_idx.reshape(_NG, _EPG) - base
    lt = to_idx.reshape(_NG, _EPG) - base

    def b2(b):
        return b.reshape(1, -1)

    weights = [
        p['enc_node'][0], b2(p['enc_node'][1]),
        p['enc_edge'][0], b2(p['enc_edge'][1]),
        p['msg1'][0], b2(p['msg1'][1]),
        p['msg2'][0], b2(p['msg2'][1]),
        p['upd1'][0], b2(p['upd1'][1]),
        p['upd2'][0], b2(p['upd2'][1]),
        p['int1'][0], b2(p['int1'][1]),
        p['int2'][0], b2(p['int2'][1]),
        p['sink1'][0], b2(p['sink1'][1]),
        p['sink2'][0], b2(p['sink2'][1]),
    ]

    grid = _NG // _G
    w_specs = [pl.BlockSpec(w.shape, lambda i: (0, 0)) for w in weights]
    out = pl.pallas_call(
        _body,
        grid=(grid,),
        in_specs=[
            pl.BlockSpec((_NB, n_feat), lambda i: (i, 0)),
            pl.BlockSpec((_EB, edge_features.shape[1]), lambda i: (i, 0)),
            pl.BlockSpec((_G, _EPG), lambda i: (i, 0)),
            pl.BlockSpec((_G, _EPG), lambda i: (i, 0)),
        ] + w_specs,
        out_specs=pl.BlockSpec((_P, 1), lambda i: (i, 0)),
        out_shape=jax.ShapeDtypeStruct((_B, 1), _F32),
    )(nfp, edge_features, lf, lt, *weights)
    return out.reshape(_B)


# speedup vs baseline: 1.1270x; 1.0374x over previous
import jax
import jax.numpy as jnp
from jax.experimental import pallas as pl

_B = 256
_NPG = 30
_EPG = 96
_MAXE = 128
_NG = 2 * _B
_NODE_FEAT = 16
_STATE = 32
_MSG = 32
_PROP = 3
_TIME = 2
_SINK_ITERS = 10
_TEMP = 0.1

_NP = 32
_P = 8
_G = 2 * _P
_EB = _G * _EPG
_NB = _G * _NP
_F32 = jnp.float32
_LOG2E = 1.4426950408889634


def _mm(a, b):
    return jax.lax.dot_general(a, b, (((1,), (0,)), ((), ())),
                               preferred_element_type=_F32)


_mm32 = _mm


def _bmm(a, b):
    return jax.lax.dot_general(a, b, (((2,), (1,)), ((0,), (0,))),
                               preferred_element_type=_F32)


def _body(nf_ref, ef_ref, lf_ref, lt_ref,
          enw, enb, eew, eeb,
          m1w, m1b, m2w, m2b,
          u1w, u1b, u2w, u2b,
          i1w, i1b, i2w, i2b,
          s1w, s1b, s2w, s2b,
          out_ref):
    relu = lambda x: jnp.maximum(x, 0.0)

    lf = lf_ref[...].reshape(_G, _EPG, 1)
    lt = lt_ref[...].reshape(_G, _EPG, 1)
    i2n = jax.lax.broadcasted_iota(jnp.int32, (_G, _EPG, 2 * _NP), 2)
    s_cat = jnp.logical_or(lf == i2n, lt + _NP == i2n).astype(_F32)
    i1n = jax.lax.broadcasted_iota(jnp.int32, (_G, _EPG, _NP), 2)
    s_to = (lt == i1n).astype(_F32)

    h0 = _mm(nf_ref[...], enw[...]) + enb[...]
    e0 = _mm(ef_ref[...], eew[...]) + eeb[...]

    m1w_v = m1w[...]
    w1h = jnp.concatenate([m1w_v[0:_STATE, :], m1w_v[_STATE:2 * _STATE, :]],
                          axis=1)
    wc = _mm32(i2w[...], m1w_v[2 * _STATE:, :])
    bc = _mm32(i2b[...], m1w_v[2 * _STATE:, :]) + m1b[...]
    i1w_v = i1w[...]
    m2w_v = m2w[...]
    m2b_v = m2b[...]

    def hcat(h):
        hab = _mm(h, w1h)
        h_a = hab[:, 0:2 * _STATE].reshape(_G, _NP, 2 * _STATE)
        h_b = hab[:, 2 * _STATE:].reshape(_G, _NP, 2 * _STATE)
        return jnp.concatenate([h_a, h_b], axis=1)

    def gath_relu(hc, c_c):
        gath = _bmm(s_cat, hc).reshape(_EB, 2 * _STATE)
        return relu(gath + c_c)

    m2i = _mm(m2w_v, i1w_v[0:_MSG, :])
    b2i = _mm(m2b_v, i1w_v[0:_MSG, :])
    x10 = _mm(e0, i1w_v[0:_MSG, :])
    hc0 = hcat(h0)

    def node_update(h, hc, c_c):
        r1 = jnp.concatenate(
            [gath_relu(hc, c_c), jnp.ones((_EB, 1), _F32)], axis=1)
        aggc = jax.lax.dot_general(
            s_to,
            r1.reshape(_G, _EPG, 2 * _STATE + 1),
            (((1,), (1,)), ((0,), (0,))),
            preferred_element_type=_F32).reshape(_NB, 2 * _STATE + 1)
        agg = (_mm(aggc[:, 0:2 * _STATE], m2w_v)
               + aggc[:, 2 * _STATE:] * m2b_v)
        hu = jnp.concatenate([h, agg], axis=1)
        h = _mm(relu(_mm(hu, u1w[...]) + u1b[...]), u2w[...]) + u2b[...]
        return h, hcat(h)

    def make_inter(efe, tp):
        efe4 = efe.reshape(_P, 2, _EPG, _MSG)
        zpad = jnp.zeros((_P, _MAXE - _EPG, _MSG), _F32)
        qp = jnp.concatenate([efe4[:, 0], zpad], axis=1)
        cp = jnp.concatenate([efe4[:, 1], zpad], axis=1)
        qi = _bmm(tp, cp)
        ci = jax.lax.dot_general(
            tp, qp, (((1,), (1,)), ((0,), (0,))),
            preferred_element_type=_F32)
        return jnp.concatenate(
            [qi[:, 0:_EPG, :].reshape(_P, 1, _EPG, _MSG),
             ci[:, 0:_EPG, :].reshape(_P, 1, _EPG, _MSG)],
            axis=1).reshape(_EB, _MSG)

    c_c0 = _mm(relu(x10 + i1b[...]), wc) + bc
    h1, hc1 = node_update(h0, hc0, c_c0)
    efe1 = _mm(gath_relu(hc1, c_c0), m2w_v) + m2b_v
    x1e = _mm(efe1, i1w_v[0:_MSG, :])

    transport_plan = None
    qs = cs = None
    for _t in range(_TIME):
        h, hc, efe = h1, hc1, efe1
        x1 = x1e
        inter = None
        if transport_plan is not None:
            inter = make_inter(efe1, transport_plan)
        for _s in range(1, _PROP):
            if inter is not None:
                x1 = x1 + _mm(inter, i1w_v[_MSG:, :])
            c_c = _mm(relu(x1 + i1b[...]), wc) + bc
            h, hc = node_update(h, hc, c_c)
            pre2 = gath_relu(hc, c_c)

            if transport_plan is None and _s + 1 < _PROP:
                x1 = _mm(pre2, m2i) + b2i
            else:
                efe = _mm(pre2, m2w_v) + m2b_v
                if _s + 1 < _PROP:
                    x1 = _mm(efe, i1w_v[0:_MSG, :])

            if transport_plan is not None and _s + 1 < _PROP:
                inter = make_inter(efe, transport_plan)

        efe4 = efe.reshape(_P, 2, _EPG, _MSG)
        zpad = jnp.zeros((_P, _MAXE - _EPG, _MSG), _F32)
        qs = jnp.concatenate([efe4[:, 0], zpad], axis=1)
        cs = jnp.concatenate([efe4[:, 1], zpad], axis=1)

        def sink_mlp(x):
            y = _mm32(relu(_mm32(x.reshape(_P * _MAXE, _MSG), s1w[...])
                           + s1b[...]), s2w[...]) + s2b[...]
            return y.reshape(_P, _MAXE, _MSG)

        rowmask = (jax.lax.broadcasted_iota(jnp.int32, (_P, _MAXE, 1), 1)
                   < _EPG).astype(_F32)
        tq = sink_mlp(qs) * rowmask
        tc = sink_mlp(cs) * rowmask
        la0 = jax.lax.dot_general(
            tq, tc, (((2,), (2,)), ((0,), (0,))),
            preferred_element_type=_F32) * (_LOG2E / _TEMP)
        ones_col = jnp.ones((_MAXE, 1), _F32)
        f = jnp.max(la0, axis=2, keepdims=True)
        e1 = jnp.exp2(la0 - f)
        s1 = jax.lax.dot_general(
            e1, ones_col, (((2,), (0,)), ((), ())),
            preferred_element_type=_F32)
        f = f + jnp.log2(s1)
        x2 = la0 - f
        m2 = jnp.max(x2, axis=1, keepdims=True)
        e2 = jnp.exp2(x2 - m2)
        s2 = jnp.sum(e2, axis=1, keepdims=True)
        r2 = 1.0 / s2
        for _i in range(_SINK_ITERS - 1):
            e1 = e2 * r2
            s1 = jax.lax.dot_general(
                e1, ones_col, (((2,), (0,)), ((), ())),
                preferred_element_type=_F32)
            e2 = e1 * (1.0 / s1)
            s2 = jnp.sum(e2, axis=1, keepdims=True)
            r2 = 1.0 / s2
        transport_plan = e2 * r2

    tpc = jax.lax.dot_general(
        transport_plan, cs, (((2,), (1,)), ((0,), (0,))),
        preferred_element_type=_F32)
    score = -jnp.sum(relu(qs - tpc), axis=(1, 2))
    out_ref[...] = score.reshape(_P, 1)


def kernel(node_features, edge_features, params, from_idx, to_idx):
    p = params
    n_feat = node_features.shape[1]

    nfp = jnp.pad(node_features.reshape(_NG, _NPG, n_feat),
                  ((0, 0), (0, _NP - _NPG), (0, 0))).reshape(_NG * _NP, n_feat)
    base = (jnp.arange(_NG, dtype=jnp.int32) * _NPG)[:, None]
    lf = from_idx.reshape(_NG, _EPG) - base
    lt = to_idx.reshape(_NG, _EPG) - base

    def b2(b):
        return b.reshape(1, -1)

    weights = [
        p['enc_node'][0], b2(p['enc_node'][1]),
        p['enc_edge'][0], b2(p['enc_edge'][1]),
        p['msg1'][0], b2(p['msg1'][1]),
        p['msg2'][0], b2(p['msg2'][1]),
        p['upd1'][0], b2(p['upd1'][1]),
        p['upd2'][0], b2(p['upd2'][1]),
        p['int1'][0], b2(p['int1'][1]),
        p['int2'][0], b2(p['int2'][1]),
        p['sink1'][0], b2(p['sink1'][1]),
        p['sink2'][0], b2(p['sink2'][1]),
    ]

    grid = _NG // _G
    w_specs = [pl.BlockSpec(w.shape, lambda i: (0, 0)) for w in weights]
    out = pl.pallas_call(
        _body,
        grid=(grid,),
        in_specs=[
            pl.BlockSpec((_NB, n_feat), lambda i: (i, 0)),
            pl.BlockSpec((_EB, edge_features.shape[1]), lambda i: (i, 0)),
            pl.BlockSpec((_G, _EPG), lambda i: (i, 0)),
            pl.BlockSpec((_G, _EPG), lambda i: (i, 0)),
        ] + w_specs,
        out_specs=pl.BlockSpec((_P, 1), lambda i: (i, 0)),
        out_shape=jax.ShapeDtypeStruct((_B, 1), _F32),
    )(nfp, edge_features, lf, lt, *weights)
    return out.reshape(_B)
